# baseline (speedup 1.0000x reference)
import functools

import jax
import jax.numpy as jnp
from jax import lax
from jax.experimental import pallas as pl
from jax.experimental.pallas import tpu as pltpu
from jax.experimental.pallas import tpu_sc as plsc

D = 32
NC, NS = 2, 16
NW = NC * NS
BATCH = 16384
SEQ = 50
BPW = BATCH // NW
NBB = BPW // 128
UNITS = NBB * SEQ
NB = 4
SEQP = 56
OPITCH = 137
TROWS = 250000


def _advance(s, bb):
    wrap = s == SEQ - 1
    return jnp.where(wrap, 0, s + 1), bb + wrap.astype(jnp.int32)


def _xstage_body(xt, out, xs_v):
    wid = lax.axis_index("s") * NC + lax.axis_index("c")
    for j in range(NBB):
        pltpu.sync_copy(xt.at[:, pl.ds(wid * BPW + j * 128, 128)],
                        xs_v.at[pl.ds(j * SEQ, SEQ)])
    pltpu.sync_copy(xs_v, out.at[pl.ds(wid * UNITS, UNITS)])


_xstage = functools.partial(
    pl.kernel,
    out_type=jax.ShapeDtypeStruct((NW * UNITS, 128), jnp.int32),
    mesh=plsc.VectorSubcoreMesh(
        core_axis_name="c", subcore_axis_name="s",
        num_cores=NC, num_subcores=NS),
    scratch_types=[pltpu.VMEM((UNITS, 128), jnp.int32)],
    compiler_params=pltpu.CompilerParams(needs_layout_passes=False),
)(_xstage_body)


def _emb_body(tbl, xt, out, xstage, idx4, rows4, obuf,
              gsem0, gsem1, gsem2, gsem3, wsem0, wsem1, wsem2, wsem3):
    wid = lax.axis_index("s") * NC + lax.axis_index("c")
    b0w = wid * BPW
    gsems = (gsem0, gsem1, gsem2, gsem3)
    wsems = (wsem0, wsem1, wsem2, wsem3)

    pltpu.sync_copy(xt.at[pl.ds(wid * UNITS, UNITS)], xstage)

    iota = lax.iota(jnp.int32, 16)

    def prep(slot, s, bb):
        for l in range(8):
            idx4[slot, pl.ds(l * 16, 16)] = xstage[bb * SEQ + s, pl.ds(l * 16, 16)]

    def gather_desc(slot):
        return pltpu.make_async_copy(
            tbl.at[idx4.at[slot]], rows4.at[slot], gsems[slot])

    def write_desc(slot, s, bb):
        return pltpu.make_async_copy(
            obuf.at[slot, 0, :, pl.ds(0, 128)],
            out.at[pl.ds(s * D, D), pl.ds(b0w + bb * 128, 128)],
            wsems[slot])

    for u in range(NB):
        prep(u, u, 0)
        gather_desc(u).start()

    def step(q, c):
        s_d, bb_d, s_f, bb_f, s_w, bb_w = c
        gather_desc(q).wait()

        @pl.when(bb_w >= 0)
        def _():
            write_desc(q, s_w, bb_w).wait()

        obuf_q = obuf.at[q, 0]
        iota16 = iota + 16

        def bgroup(g, carry):
            for j in range(16):
                b = g * 16 + j
                v0 = rows4[q, b, pl.ds(0, 16)]
                v1 = rows4[q, b, pl.ds(16, 16)]
                bvec = jnp.full((16,), b, jnp.int32)
                plsc.store_scatter(obuf_q, [iota, bvec], v0)
                plsc.store_scatter(obuf_q, [iota16, bvec], v1)
            return carry
        lax.fori_loop(0, 8, bgroup, 0)

        write_desc(q, s_d, bb_d).start()

        @pl.when(bb_f < NBB)
        def _():
            prep(q, s_f, bb_f)
            gather_desc(q).start()

        s_d, bb_d = _advance(s_d, bb_d)
        s_f, bb_f = _advance(s_f, bb_f)
        s_w, bb_w = _advance(s_w, bb_w)
        return s_d, bb_d, s_f, bb_f, s_w, bb_w

    def body(t, c):
        for q in range(NB):
            c = step(q, c)
        return c

    c0 = (jnp.int32(0), jnp.int32(0),
          jnp.int32(NB), jnp.int32(0),
          jnp.int32(SEQ - NB), jnp.int32(-1))
    c = lax.fori_loop(0, UNITS // NB, body, c0)

    s_w, bb_w = c[4], c[5]
    for q in range(NB):
        write_desc(q, s_w, bb_w).wait()
        s_w, bb_w = _advance(s_w, bb_w)


_emb = functools.partial(
    pl.kernel,
    out_type=jax.ShapeDtypeStruct((SEQ * D, BATCH), jnp.float32),
    mesh=plsc.VectorSubcoreMesh(
        core_axis_name="c", subcore_axis_name="s",
        num_cores=NC, num_subcores=NS),
    scratch_types=[
        pltpu.VMEM((UNITS, 128), jnp.int32),
        pltpu.VMEM((NB, 128), jnp.int32),
        pltpu.VMEM((NB, 128, D), jnp.float32),
        pltpu.VMEM((NB, 1, D, OPITCH), jnp.float32),
        pltpu.SemaphoreType.DMA,
        pltpu.SemaphoreType.DMA,
        pltpu.SemaphoreType.DMA,
        pltpu.SemaphoreType.DMA,
        pltpu.SemaphoreType.DMA,
        pltpu.SemaphoreType.DMA,
        pltpu.SemaphoreType.DMA,
        pltpu.SemaphoreType.DMA,
    ],
    compiler_params=pltpu.CompilerParams(
        needs_layout_passes=False, use_tc_tiling_on_sc=False),
)(_emb_body)


@jax.jit
def kernel(x, table):
    xt = jnp.transpose(x.astype(jnp.int32))
    idxall = _xstage(xt)
    out_t = _emb(table, idxall)
    out3 = jnp.reshape(out_t, (SEQ, D, BATCH))
    return jnp.transpose(out3, (2, 0, 1))
